# baseline (speedup 1.0000x reference)
import functools

import jax
import jax.numpy as jnp
from jax import lax
from jax.experimental import pallas as pl
from jax.experimental.pallas import tpu as pltpu
from jax.experimental.pallas import tpu_sc as plsc

_B = 4
_S = 2048
_D = 1024
_N = _B * _S

_info = plsc.get_sparse_core_info()
_NC = _info.num_cores
_NS = _info.num_subcores
_NW = _NC * _NS
_ROWS_PER_W = _N // _NW
_C = 32
_NCHUNK = _ROWS_PER_W // _C


def _make_sc_kernel():
    mesh = plsc.VectorSubcoreMesh(core_axis_name="c", subcore_axis_name="s")

    @functools.partial(
        pl.kernel,
        mesh=mesh,
        out_type=jax.ShapeDtypeStruct((_N, _D), jnp.float32),
        scratch_types=[
            pltpu.VMEM((_ROWS_PER_W,), jnp.int32),
            pltpu.VMEM((_C, _D), jnp.float32),
            pltpu.VMEM((_C, _D), jnp.float32),
            pltpu.SemaphoreType.DMA,
        ],
    )
    def k(table_hbm, idx_hbm, pos_hbm, out_hbm, idx_v, rows_v, pos_v, sem):
        wid = lax.axis_index("s") * _NC + lax.axis_index("c")
        base = wid * _ROWS_PER_W
        s_base = lax.rem(base, _S)
        pltpu.sync_copy(idx_hbm.at[pl.ds(base, _ROWS_PER_W)], idx_v)
        for j in range(_NCHUNK):
            gcopy = pltpu.make_async_copy(
                table_hbm.at[idx_v.at[pl.ds(j * _C, _C)]], rows_v, sem
            )
            gcopy.start()
            pltpu.sync_copy(pos_hbm.at[pl.ds(s_base + j * _C, _C)], pos_v)
            gcopy.wait()
            for r in range(_C):

                def add_body(c, _, r=r):
                    sl = pl.ds(c * 16, 16)
                    rows_v[r, sl] = rows_v[r, sl] + pos_v[r, sl]
                    return 0

                lax.fori_loop(0, _D // 16, add_body, 0)
            pltpu.sync_copy(rows_v, out_hbm.at[pl.ds(base + j * _C, _C)])

    return k


_sc_kernel = _make_sc_kernel()


@jax.jit
def kernel(x, table, pos_enc):
    x_flat = x.reshape(-1)
    pos = pos_enc[:_S]
    out = _sc_kernel(table, x_flat, pos)
    return out.reshape(_B, _S, _D)


# speedup vs baseline: 2.4678x; 2.4678x over previous
import functools

import jax
import jax.numpy as jnp
from jax import lax
from jax.experimental import pallas as pl
from jax.experimental.pallas import tpu as pltpu
from jax.experimental.pallas import tpu_sc as plsc

_B = 4
_S = 2048
_D = 1024
_N = _B * _S

_info = plsc.get_sparse_core_info()
_NC = _info.num_cores
_NS = _info.num_subcores
_NW = _NC * _NS
_SPW = _S // _NW
_C = 16
_NCHUNK = (_B * _SPW) // _C
_HPB = _SPW // _C
_NB = 3


def _make_sc_kernel():
    mesh = plsc.VectorSubcoreMesh(core_axis_name="c", subcore_axis_name="s")

    @functools.partial(
        pl.kernel,
        mesh=mesh,
        out_type=jax.ShapeDtypeStruct((_N, _D), jnp.float32),
        scratch_types=[
            pltpu.VMEM((_B * _SPW,), jnp.int32),
            pltpu.VMEM((_SPW, _D), jnp.float32),
            *[pltpu.VMEM((_C, _D), jnp.float32) for _ in range(_NB)],
            pltpu.SemaphoreType.DMA,
            pltpu.SemaphoreType.DMA,
        ],
    )
    def k(table_hbm, idx_hbm, pos_hbm, out_hbm, idx_v, pos_v, *bufs_and_sems):
        bufs = bufs_and_sems[:_NB]
        gsem, ssem = bufs_and_sems[_NB:]
        wid = lax.axis_index("s") * _NC + lax.axis_index("c")
        s0 = wid * _SPW
        for b in range(_B):
            pltpu.sync_copy(
                idx_hbm.at[pl.ds(b * _S + s0, _SPW)],
                idx_v.at[pl.ds(b * _SPW, _SPW)],
            )
        pltpu.sync_copy(pos_hbm.at[pl.ds(s0, _SPW)], pos_v)

        def store_desc(t):
            b, h = divmod(t, _HPB)
            return pltpu.make_async_copy(
                bufs[t % _NB],
                out_hbm.at[pl.ds(b * _S + s0 + h * _C, _C)],
                ssem,
            )

        for t in range(_NCHUNK + 1):
            if t < _NCHUNK:
                if t >= _NB:
                    store_desc(t - _NB).wait()
                pltpu.async_copy(
                    table_hbm.at[idx_v.at[pl.ds(t * _C, _C)]],
                    bufs[t % _NB],
                    gsem,
                )
            if t >= 1:
                u = t - 1
                buf = bufs[u % _NB]
                pltpu.make_async_copy(
                    table_hbm.at[idx_v.at[pl.ds(u * _C, _C)]], buf, gsem
                ).wait()
                h = u % _HPB

                @plsc.parallel_loop(0, (_C * _D) // 16, unroll=8)
                def add_body(i, _h=h, _buf=buf):
                    r = lax.shift_right_logical(i, 6)
                    sl = pl.ds((i & 63) * 16, 16)
                    _buf[r, sl] = _buf[r, sl] + pos_v[_h * _C + r, sl]

                store_desc(u).start()
        for t in range(_NCHUNK - _NB, _NCHUNK):
            store_desc(t).wait()

    return k


_sc_kernel = _make_sc_kernel()


@jax.jit
def kernel(x, table, pos_enc):
    x_flat = x.reshape(-1)
    pos = pos_enc[:_S]
    out = _sc_kernel(table, x_flat, pos)
    return out.reshape(_B, _S, _D)


# speedup vs baseline: 2.7493x; 1.1141x over previous
import jax
import jax.numpy as jnp
from jax import lax
from jax.experimental import pallas as pl
from jax.experimental.pallas import tpu as pltpu
from jax.experimental.pallas import tpu_sc as plsc

_B = 4
_S = 2048
_D = 1024

_info = plsc.get_sparse_core_info()
_NC = _info.num_cores
_NS = _info.num_subcores
_NW = _NC * _NS
_SPW = _S // _NW
_C = 16
_NCHUNK = (_B * _SPW) // _C
_HPB = _SPW // _C
_NB = 3


def _make_sc_kernel():
    mesh = plsc.VectorSubcoreMesh(core_axis_name="c", subcore_axis_name="s")

    def kfn(table_hbm, x_hbm, pos_hbm, out_hbm, idx_v, pos_v, *bufs_and_sems):
        bufs = bufs_and_sems[:_NB]
        gsem, ssem = bufs_and_sems[_NB:]
        wid = lax.axis_index("s") * _NC + lax.axis_index("c")
        s0 = wid * _SPW
        for b in range(_B):
            pltpu.sync_copy(
                x_hbm.at[b, pl.ds(s0, _SPW)],
                idx_v.at[pl.ds(b * _SPW, _SPW)],
            )
        pltpu.sync_copy(pos_hbm.at[pl.ds(s0, _SPW)], pos_v)

        def store_desc(t):
            b, h = divmod(t, _HPB)
            return pltpu.make_async_copy(
                bufs[t % _NB],
                out_hbm.at[b, pl.ds(s0 + h * _C, _C)],
                ssem,
            )

        for t in range(_NCHUNK + 1):
            if t < _NCHUNK:
                if t >= _NB:
                    store_desc(t - _NB).wait()
                pltpu.async_copy(
                    table_hbm.at[idx_v.at[pl.ds(t * _C, _C)]],
                    bufs[t % _NB],
                    gsem,
                )
            if t >= 1:
                u = t - 1
                buf = bufs[u % _NB]
                pltpu.make_async_copy(
                    table_hbm.at[idx_v.at[pl.ds(u * _C, _C)]], buf, gsem
                ).wait()
                h = u % _HPB

                @plsc.parallel_loop(0, (_C * _D) // 16, unroll=8)
                def add_body(i, _h=h, _buf=buf):
                    r = lax.shift_right_logical(i, 6)
                    sl = pl.ds((i & 63) * 16, 16)
                    _buf[r, sl] = _buf[r, sl] + pos_v[_h * _C + r, sl]

                store_desc(u).start()
        for t in range(_NCHUNK - _NB, _NCHUNK):
            store_desc(t).wait()

    return pl.kernel(
        kfn,
        mesh=mesh,
        out_type=jax.ShapeDtypeStruct((_B, _S, _D), jnp.float32),
        scratch_types=[
            pltpu.VMEM((_B * _SPW,), jnp.int32),
            pltpu.VMEM((_SPW, _D), jnp.float32),
            *[pltpu.VMEM((_C, _D), jnp.float32) for _ in range(_NB)],
            pltpu.SemaphoreType.DMA,
            pltpu.SemaphoreType.DMA,
        ],
    )


_sc_kernel = _make_sc_kernel()


@jax.jit
def kernel(x, table, pos_enc):
    return _sc_kernel(table, x, pos_enc)


# speedup vs baseline: 2.8425x; 1.0339x over previous
import jax
import jax.numpy as jnp
from jax import lax
from jax.experimental import pallas as pl
from jax.experimental.pallas import tpu as pltpu
from jax.experimental.pallas import tpu_sc as plsc

_B = 4
_S = 2048
_D = 1024

_info = plsc.get_sparse_core_info()
_NC = _info.num_cores
_NS = _info.num_subcores
_NW = _NC * _NS
_SPW = _S // _NW
_C = 8
_NSUP = _SPW // _C
_NG = 3


def _make_sc_kernel():
    mesh = plsc.VectorSubcoreMesh(core_axis_name="c", subcore_axis_name="s")

    def kfn(table_hbm, x_hbm, pos_hbm, out_hbm, idx_v, *rest):
        nbuf = _NG * (1 + _B)
        bufs = rest[:nbuf]
        gsems = rest[nbuf : nbuf + _NG]
        ssems = rest[nbuf + _NG :]
        pbuf = lambda gg: bufs[gg * (1 + _B)]
        abuf = lambda gg, b: bufs[gg * (1 + _B) + 1 + b]

        wid = lax.axis_index("s") * _NC + lax.axis_index("c")
        s0 = wid * _SPW
        for b in range(_B):
            pltpu.sync_copy(
                x_hbm.at[b, pl.ds(s0, _SPW)],
                idx_v.at[pl.ds(b * _SPW, _SPW)],
            )

        def gather_descs(g):
            gg = g % _NG
            descs = [
                pltpu.make_async_copy(
                    pos_hbm.at[pl.ds(s0 + g * _C, _C)], pbuf(gg), gsems[gg]
                )
            ]
            for b in range(_B):
                descs.append(
                    pltpu.make_async_copy(
                        table_hbm.at[idx_v.at[pl.ds(b * _SPW + g * _C, _C)]],
                        abuf(gg, b),
                        gsems[gg],
                    )
                )
            return descs

        def store_descs(g):
            gg = g % _NG
            return [
                pltpu.make_async_copy(
                    abuf(gg, b),
                    out_hbm.at[b, pl.ds(s0 + g * _C, _C)],
                    ssems[gg],
                )
                for b in range(_B)
            ]

        def issue(g):
            if g >= _NG:
                for d in store_descs(g - _NG):
                    d.wait()
            for d in gather_descs(g):
                d.start()

        def consume(g):
            gg = g % _NG
            for d in gather_descs(g):
                d.wait()
            p, bs = pbuf(gg), [abuf(gg, b) for b in range(_B)]

            @plsc.parallel_loop(0, (_C * _D) // 16, unroll=4)
            def add_body(i, _p=p, _bs=bs):
                r = lax.shift_right_logical(i, 6)
                sl = pl.ds((i & 63) * 16, 16)
                pv = _p[r, sl]
                for buf in _bs:
                    buf[r, sl] = buf[r, sl] + pv

            for d in store_descs(g):
                d.start()

        issue(0)
        issue(1)
        for g in range(_NSUP):
            if g + 2 < _NSUP:
                issue(g + 2)
            consume(g)
        for g in range(_NSUP - _NG, _NSUP):
            for d in store_descs(g):
                d.wait()

    return pl.kernel(
        kfn,
        mesh=mesh,
        out_type=jax.ShapeDtypeStruct((_B, _S, _D), jnp.float32),
        scratch_types=[
            pltpu.VMEM((_B * _SPW,), jnp.int32),
            *[
                pltpu.VMEM((_C, _D), jnp.float32)
                for _ in range(_NG * (1 + _B))
            ],
            *[pltpu.SemaphoreType.DMA for _ in range(2 * _NG)],
        ],
    )


_sc_kernel = _make_sc_kernel()


@jax.jit
def kernel(x, table, pos_enc):
    return _sc_kernel(table, x, pos_enc)
